# Initial kernel scaffold; baseline (speedup 1.0000x reference)
import functools

import jax
import jax.numpy as jnp
from jax import lax
from jax.experimental import pallas as pl
from jax.experimental.pallas import tpu as pltpu
from jax.experimental.pallas import tpu_sc as plsc

D = 16
NC = 2
NS = 16
NW = NC * NS
L = 16
R_CONST = 2.0
T_CONST = 1.0


def _tc_precompute_body(h_ref, wa_ref, wb_ref, bd_ref, u_ref, v_ref):
    h = h_ref[...]
    nrm = jnp.sqrt(jnp.sum(h * h, axis=1, keepdims=True))
    hn = h / (nrm + 1e-12)
    u_ref[...] = jnp.dot(hn, wa_ref[...], preferred_element_type=jnp.float32)
    v_ref[...] = (
        jnp.dot(hn, wb_ref[...], preferred_element_type=jnp.float32)
        + bd_ref[0:1, :]
    )


@functools.lru_cache(maxsize=None)
def _make_tc_precompute(n_nodes: int):
    rows = 10000
    assert n_nodes % rows == 0
    grid = n_nodes // rows
    return pl.pallas_call(
        _tc_precompute_body,
        grid=(grid,),
        in_specs=[
            pl.BlockSpec((rows, D), lambda i: (i, 0)),
            pl.BlockSpec((D, D), lambda i: (0, 0)),
            pl.BlockSpec((D, D), lambda i: (0, 0)),
            pl.BlockSpec((8, D), lambda i: (0, 0)),
        ],
        out_specs=[
            pl.BlockSpec((rows, D), lambda i: (i, 0)),
            pl.BlockSpec((rows, D), lambda i: (i, 0)),
        ],
        out_shape=[
            jax.ShapeDtypeStruct((n_nodes, D), jnp.float32),
            jax.ShapeDtypeStruct((n_nodes, D), jnp.float32),
        ],
    )


@functools.lru_cache(maxsize=None)
def _make_sc_edges(n_edges: int, chunk: int):
    assert n_edges % NW == 0
    epw = n_edges // NW
    assert epw % chunk == 0 and chunk % L == 0
    nchunk = epw // chunk
    mesh = plsc.VectorSubcoreMesh(
        core_axis_name="c", subcore_axis_name="s", num_cores=NC, num_subcores=NS
    )

    @functools.partial(
        pl.kernel,
        mesh=mesh,
        out_type=jax.ShapeDtypeStruct((n_edges,), jnp.float32),
        scratch_types=[
            pltpu.VMEM((chunk,), jnp.int32),
            pltpu.VMEM((chunk,), jnp.int32),
            pltpu.VMEM((chunk, D), jnp.float32),
            pltpu.VMEM((chunk, D), jnp.float32),
            pltpu.VMEM((chunk,), jnp.float32),
            pltpu.SemaphoreType.DMA,
        ],
    )
    def edge_kernel(u_hbm, v_hbm, src_hbm, dst_hbm, out_hbm,
                    src_v, dst_v, urows, vrows, out_v, sem):
        wid = lax.axis_index("s") * NC + lax.axis_index("c")
        base = wid * epw
        lane = lax.iota(jnp.int32, L)

        def chunk_body(g, carry):
            off = base + g * chunk
            pltpu.sync_copy(src_hbm.at[pl.ds(off, chunk)], src_v)
            pltpu.sync_copy(dst_hbm.at[pl.ds(off, chunk)], dst_v)
            cu = pltpu.async_copy(u_hbm.at[src_v], urows, sem)
            cv = pltpu.async_copy(v_hbm.at[dst_v], vrows, sem)
            cu.wait()
            cv.wait()

            def group_body(t, carry2):
                e0 = t * L
                rows = e0 + lane
                acc = jnp.zeros((L,), jnp.float32)
                for d in range(D):
                    dv = jnp.full((L,), d, jnp.int32)
                    xu = plsc.load_gather(urows, [rows, dv])
                    xv = plsc.load_gather(vrows, [rows, dv])
                    s = xu + xv
                    acc = acc + s * s
                z = (acc - R_CONST) / T_CONST
                z = jnp.minimum(jnp.maximum(z, -50.0), 50.0)
                p = 1.0 / (jnp.exp(z) + 1.0)
                out_v[pl.ds(e0, L)] = p
                return carry2

            lax.fori_loop(0, chunk // L, group_body, 0, unroll=False)
            pltpu.sync_copy(out_v, out_hbm.at[pl.ds(off, chunk)])
            return carry

        lax.fori_loop(0, nchunk, chunk_body, 0, unroll=False)

    return edge_kernel


def kernel(h, idx, W, b):
    n_nodes = h.shape[0]
    n_edges = idx.shape[0]
    wd = W[:16, :] - W[16:, :]
    wa = wd[:, :16].T
    wb = wd[:, 16:].T
    bd = b[:16] - b[16:]
    bd8 = jnp.tile(bd[None, :], (8, 1))
    u, v = _make_tc_precompute(n_nodes)(h, wa, wb, bd8)
    src = idx[:, 0]
    dst = idx[:, 1]
    return _make_sc_edges(n_edges, 2000)(u, v, src, dst)


# speedup vs baseline: 21.3983x; 21.3983x over previous
import functools

import jax
import jax.numpy as jnp
from jax import lax
from jax.experimental import pallas as pl
from jax.experimental.pallas import tpu as pltpu
from jax.experimental.pallas import tpu_sc as plsc

D = 16
NC = 2
NS = 16
NW = NC * NS
L = 16
R_CONST = 2.0
T_CONST = 1.0


def _tc_precompute_body(h_ref, wa_ref, wb_ref, bd_ref, u_ref, v_ref):
    h = h_ref[...]
    nrm = jnp.sqrt(jnp.sum(h * h, axis=1, keepdims=True))
    hn = h / (nrm + 1e-12)
    u_ref[...] = jnp.dot(hn, wa_ref[...], preferred_element_type=jnp.float32)
    v_ref[...] = (
        jnp.dot(hn, wb_ref[...], preferred_element_type=jnp.float32)
        + bd_ref[0:1, :]
    )


@functools.lru_cache(maxsize=None)
def _make_tc_precompute(n_nodes: int):
    rows = 10000
    assert n_nodes % rows == 0
    grid = n_nodes // rows
    return pl.pallas_call(
        _tc_precompute_body,
        grid=(grid,),
        in_specs=[
            pl.BlockSpec((rows, D), lambda i: (i, 0)),
            pl.BlockSpec((D, D), lambda i: (0, 0)),
            pl.BlockSpec((D, D), lambda i: (0, 0)),
            pl.BlockSpec((8, D), lambda i: (0, 0)),
        ],
        out_specs=[
            pl.BlockSpec((rows, D), lambda i: (i, 0)),
            pl.BlockSpec((rows, D), lambda i: (i, 0)),
        ],
        out_shape=[
            jax.ShapeDtypeStruct((n_nodes, D), jnp.float32),
            jax.ShapeDtypeStruct((n_nodes, D), jnp.float32),
        ],
    )


@functools.lru_cache(maxsize=None)
def _make_sc_edges(n_edges: int, chunk: int):
    assert n_edges % NW == 0
    epw = n_edges // NW
    assert epw % chunk == 0 and chunk % L == 0
    nchunk = epw // chunk
    mesh = plsc.VectorSubcoreMesh(
        core_axis_name="c", subcore_axis_name="s", num_cores=NC, num_subcores=NS
    )

    @functools.partial(
        pl.kernel,
        mesh=mesh,
        out_type=jax.ShapeDtypeStruct((n_edges,), jnp.float32),
        scratch_types=[
            pltpu.VMEM((chunk,), jnp.int32),
            pltpu.VMEM((chunk,), jnp.int32),
            pltpu.VMEM((chunk, D), jnp.float32),
            pltpu.VMEM((chunk, D), jnp.float32),
            pltpu.VMEM((chunk,), jnp.float32),
            pltpu.SemaphoreType.DMA,
        ],
        compiler_params=pltpu.CompilerParams(
            needs_layout_passes=False, use_tc_tiling_on_sc=False
        ),
    )
    def edge_kernel(u_hbm, v_hbm, src_hbm, dst_hbm, out_hbm,
                    src_v, dst_v, urows, vrows, out_v, sem):
        wid = lax.axis_index("s") * NC + lax.axis_index("c")
        base = wid * epw
        lane = lax.iota(jnp.int32, L)

        def chunk_body(g, carry):
            off = base + g * chunk
            pltpu.sync_copy(src_hbm.at[pl.ds(off, chunk)], src_v)
            pltpu.sync_copy(dst_hbm.at[pl.ds(off, chunk)], dst_v)
            cu = pltpu.async_copy(u_hbm.at[src_v], urows, sem)
            cv = pltpu.async_copy(v_hbm.at[dst_v], vrows, sem)
            cu.wait()
            cv.wait()

            def group_body(t, carry2):
                e0 = t * L
                rows = e0 + lane
                acc = jnp.zeros((L,), jnp.float32)
                for d in range(D):
                    dv = jnp.full((L,), d, jnp.int32)
                    xu = plsc.load_gather(urows, [rows, dv])
                    xv = plsc.load_gather(vrows, [rows, dv])
                    s = xu + xv
                    acc = acc + s * s
                z = (acc - R_CONST) / T_CONST
                z = jnp.minimum(jnp.maximum(z, -50.0), 50.0)
                p = 1.0 / (jnp.exp(z) + 1.0)
                out_v[pl.ds(e0, L)] = p
                return carry2

            lax.fori_loop(0, chunk // L, group_body, 0, unroll=False)
            pltpu.sync_copy(out_v, out_hbm.at[pl.ds(off, chunk)])
            return carry

        lax.fori_loop(0, nchunk, chunk_body, 0, unroll=False)

    return edge_kernel


def kernel(h, idx, W, b):
    n_nodes = h.shape[0]
    n_edges = idx.shape[0]
    wd = W[:16, :] - W[16:, :]
    wa = wd[:, :16].T
    wb = wd[:, 16:].T
    bd = b[:16] - b[16:]
    bd8 = jnp.tile(bd[None, :], (8, 1))
    u, v = _make_tc_precompute(n_nodes)(h, wa, wb, bd8)
    src = idx[:, 0]
    dst = idx[:, 1]
    return _make_sc_edges(n_edges, 2000)(u, v, src, dst)
